# baseline (speedup 1.0000x reference)
import functools

import jax
import jax.numpy as jnp
from jax import lax
from jax.experimental import pallas as pl
from jax.experimental.pallas import tpu as pltpu
from jax.experimental.pallas import tpu_sc as plsc


@functools.lru_cache(maxsize=None)
def _build_gather(n, vocab, d):
    info = plsc.get_sparse_core_info()
    nw = info.num_cores * info.num_subcores
    bpw = n // nw
    chunk = 1024
    n_chunks = bpw // chunk
    assert bpw % chunk == 0 and n % nw == 0

    mesh = plsc.VectorSubcoreMesh(core_axis_name="c", subcore_axis_name="s")

    @functools.partial(
        pl.kernel,
        mesh=mesh,
        out_type=jax.ShapeDtypeStruct((n, d), jnp.float32),
        scratch_types=[
            pltpu.VMEM((chunk,), jnp.int32),
            pltpu.VMEM((chunk, d), jnp.float32),
            pltpu.SemaphoreType.DMA,
        ],
        compiler_params=pltpu.CompilerParams(use_tc_tiling_on_sc=False),
    )
    def gather(idx_hbm, table_hbm, out_hbm, idx_v, rows_v, sem):
        wid = lax.axis_index("s") * info.num_cores + lax.axis_index("c")
        base = wid * bpw

        def body(g, carry):
            off = base + g * chunk
            pltpu.sync_copy(idx_hbm.at[pl.ds(off, chunk)], idx_v)
            pltpu.async_copy(table_hbm.at[idx_v], rows_v, sem).wait()
            pltpu.sync_copy(rows_v, out_hbm.at[pl.ds(off, chunk)])
            return carry

        lax.fori_loop(0, n_chunks, body, 0)

    return gather


def kernel(x, table):
    b, l = x.shape
    vocab, d = table.shape
    idx = x.astype(jnp.int32).reshape(-1)
    out = _build_gather(b * l, vocab, d)(idx, table)
    return out.reshape(b, l, d)


# speedup vs baseline: 1.0167x; 1.0167x over previous
import functools

import jax
import jax.numpy as jnp
from jax import lax
from jax.experimental import pallas as pl
from jax.experimental.pallas import tpu as pltpu
from jax.experimental.pallas import tpu_sc as plsc


@functools.lru_cache(maxsize=None)
def _build_gather(n, vocab, d):
    info = plsc.get_sparse_core_info()
    nw = info.num_cores * info.num_subcores
    bpw = n // nw
    chunk = 800
    n_chunks = bpw // chunk
    assert bpw % chunk == 0 and n % nw == 0 and n_chunks % 2 == 0

    mesh = plsc.VectorSubcoreMesh(core_axis_name="c", subcore_axis_name="s")

    @functools.partial(
        pl.kernel,
        mesh=mesh,
        out_type=jax.ShapeDtypeStruct((n, d), jnp.float32),
        scratch_types=[
            pltpu.VMEM((chunk,), jnp.int32),
            pltpu.VMEM((chunk,), jnp.int32),
            pltpu.VMEM((chunk, d), jnp.float32),
            pltpu.VMEM((chunk, d), jnp.float32),
            pltpu.SemaphoreType.DMA,
            pltpu.SemaphoreType.DMA,
            pltpu.SemaphoreType.DMA,
            pltpu.SemaphoreType.DMA,
        ],
        compiler_params=pltpu.CompilerParams(use_tc_tiling_on_sc=False),
    )
    def gather(idx_hbm, table_hbm, out_hbm, idx0, idx1, rows0, rows1,
               sg0, sg1, sw0, sw1):
        wid = lax.axis_index("s") * info.num_cores + lax.axis_index("c")
        base = wid * bpw
        bufs = ((idx0, rows0, sg0, sw0), (idx1, rows1, sg1, sw1))

        def start_gather(g, idx_v, rows_v, sg):
            pltpu.sync_copy(idx_hbm.at[pl.ds(base + g * chunk, chunk)], idx_v)
            pltpu.async_copy(table_hbm.at[idx_v], rows_v, sg)

        def wait_gather(idx_v, rows_v, sg):
            pltpu.make_async_copy(table_hbm.at[idx_v], rows_v, sg).wait()

        def start_wb(g, rows_v, sw):
            pltpu.async_copy(rows_v, out_hbm.at[pl.ds(base + g * chunk, chunk)], sw)

        def wait_wb(g, rows_v, sw):
            pltpu.make_async_copy(rows_v, out_hbm.at[pl.ds(base + g * chunk, chunk)], sw).wait()

        for b, (idx_v, rows_v, sg, sw) in enumerate(bufs):
            start_gather(b, idx_v, rows_v, sg)

        def body(i, carry):
            g0 = 2 * i
            for b, (idx_v, rows_v, sg, sw) in enumerate(bufs):
                wait_gather(idx_v, rows_v, sg)
                start_wb(g0 + b, rows_v, sw)
            for b, (idx_v, rows_v, sg, sw) in enumerate(bufs):
                wait_wb(g0 + b, rows_v, sw)
                start_gather(g0 + b + 2, idx_v, rows_v, sg)
            return carry

        lax.fori_loop(0, n_chunks // 2 - 1, body, 0)

        gl = n_chunks - 2
        for b, (idx_v, rows_v, sg, sw) in enumerate(bufs):
            wait_gather(idx_v, rows_v, sg)
            start_wb(gl + b, rows_v, sw)
        for b, (idx_v, rows_v, sg, sw) in enumerate(bufs):
            wait_wb(gl + b, rows_v, sw)

    return gather


def kernel(x, table):
    b, l = x.shape
    vocab, d = table.shape
    idx = x.astype(jnp.int32).reshape(-1)
    out = _build_gather(b * l, vocab, d)(idx, table)
    return out.reshape(b, l, d)
